# baseline (speedup 1.0000x reference)
import functools
import math

import ml_dtypes
import numpy as np

import jax
import jax.numpy as jnp
from jax.experimental import pallas as pl
from jax.experimental.pallas import tpu as pltpu

_BT = 512


def _masks1():
    m = np.zeros((9, 96, 512), np.float32)
    for dy in range(3):
        for dx in range(3):
            for w_out in range(28):
                w_in = w_out + dx - 1
                if 0 <= w_in < 28:
                    col = (w_out % 2) * 256 + (w_out // 2) * 16
                    m[dy * 3 + dx, dy * 32 + w_in, col:col + 16] = 1.0
    return m.reshape(864, 512).astype(ml_dtypes.bfloat16)


def _masks2():
    m = np.zeros((9, 768, 512), np.float32)
    for dy in range(3):
        for dx in range(3):
            for w_out in range(14):
                w_in = w_out + dx - 1
                if 0 <= w_in < 14:
                    col = (w_out % 2) * 256 + (w_out // 2) * 32
                    r = dy * 256 + w_in * 16
                    m[dy * 3 + dx, r:r + 16, col:col + 32] = 1.0
    return m.reshape(6912, 512).astype(ml_dtypes.bfloat16)


_M1 = _masks1()
_M2 = _masks2()


def _prep_body(c1w_ref, c2w_ref, f1w_ref, c1b_ref, c2b_ref, m1_ref, m2_ref,
               t1_ref, t2_ref, w1_ref, b1_ref, b2_ref):
    c1 = c1w_ref[...].astype(jnp.bfloat16)
    acc1 = jnp.zeros((96, 512), jnp.bfloat16)
    for tap in range(9):
        v = jnp.concatenate([c1[tap:tap + 1, :]] * 32, axis=1)
        acc1 = acc1 + m1_ref[96 * tap:96 * (tap + 1), :] * v
    t1_ref[...] = acc1

    c2 = c2w_ref[...].astype(jnp.bfloat16)
    acc2 = jnp.zeros((768, 512), jnp.bfloat16)
    for tap in range(9):
        blk = c2[16 * tap:16 * (tap + 1), :]
        w48 = jnp.concatenate([blk] * 48, axis=0)
        wb = jnp.concatenate([w48] * 16, axis=1)
        acc2 = acc2 + m2_ref[768 * tap:768 * (tap + 1), :] * wb
    t2_ref[...] = acc2

    f1 = f1w_ref[...].astype(jnp.bfloat16).reshape(7, 224, 128)
    f1 = jnp.concatenate([f1, jnp.zeros((7, 32, 128), jnp.bfloat16)], axis=1)
    w1_ref[...] = f1.reshape(1792, 128)

    b1_ref[...] = jnp.concatenate(
        [c1b_ref[...]] * 14 + [jnp.zeros((1, 32), jnp.float32)], axis=1)
    b2_ref[...] = jnp.concatenate(
        [c2b_ref[...]] * 7 + [jnp.zeros((1, 32), jnp.float32)], axis=1)


def _prep(conv1_w, conv2_w, fc1_w, conv1_b, conv2_b):
    return pl.pallas_call(
        _prep_body,
        out_shape=[
            jax.ShapeDtypeStruct((96, 512), jnp.bfloat16),
            jax.ShapeDtypeStruct((768, 512), jnp.bfloat16),
            jax.ShapeDtypeStruct((1792, 128), jnp.bfloat16),
            jax.ShapeDtypeStruct((1, 256), jnp.float32),
            jax.ShapeDtypeStruct((1, 256), jnp.float32),
        ],
        compiler_params=pltpu.CompilerParams(
            vmem_limit_bytes=64 * 1024 * 1024,
        ),
    )(conv1_w, conv2_w, fc1_w, conv1_b.reshape(1, 16), conv2_b.reshape(1, 32),
      jnp.asarray(_M1), jnp.asarray(_M2))


def _fused_cnn_body(x_ref, t1_ref, b1_ref, t2_ref, b2_ref, w1_ref, fb1_ref,
                    w2_ref, fb2_ref, o_ref, *, bt):
    xr = x_ref[...].astype(jnp.bfloat16)
    z32 = jnp.zeros((bt, 32), jnp.bfloat16)
    z4 = jnp.zeros((bt, 4), jnp.bfloat16)
    pieces = [z32]
    for hh in range(28):
        pieces.append(xr[:, 28 * hh:28 * hh + 28])
        pieces.append(z4)
    pieces.append(z32)
    x = jnp.concatenate(pieces, axis=1)
    b1 = b1_ref[...]
    b2 = b2_ref[...]

    p1 = []
    for i in range(14):
        ya = jnp.dot(x[:, 64 * i:64 * i + 96], t1_ref[...],
                     preferred_element_type=jnp.float32)
        yb = jnp.dot(x[:, 64 * i + 32:64 * i + 128], t1_ref[...],
                     preferred_element_type=jnp.float32)
        m = jnp.maximum(ya, yb)
        m = jnp.maximum(m[:, :256], m[:, 256:])
        p1.append(jnp.maximum(m + b1, 0.0).astype(jnp.bfloat16))

    z256 = jnp.zeros((bt, 256), jnp.bfloat16)
    p1f = jnp.concatenate([z256] + p1 + [z256], axis=1)

    feats = []
    for i in range(7):
        ya = jnp.dot(p1f[:, 512 * i:512 * i + 768], t2_ref[...],
                     preferred_element_type=jnp.float32)
        yb = jnp.dot(p1f[:, 512 * i + 256:512 * i + 1024], t2_ref[...],
                     preferred_element_type=jnp.float32)
        m = jnp.maximum(ya, yb)
        m = jnp.maximum(m[:, :256], m[:, 256:])
        feats.append(jnp.maximum(m + b2, 0.0).astype(jnp.bfloat16))

    ff = jnp.concatenate(feats, axis=1)
    h = jnp.dot(ff, w1_ref[...], preferred_element_type=jnp.float32)
    h = jnp.maximum(h + fb1_ref[...], 0.0).astype(jnp.bfloat16)
    y = (jnp.dot(h, w2_ref[...], preferred_element_type=jnp.float32)
         + fb2_ref[...])
    o_ref[...] = y[:, :10]


def kernel(x_nchw, conv1_w, conv1_b, conv2_w, conv2_b, fc1_w, fc1_b,
           fc2_w, fc2_b):
    n = x_nchw.shape[0]
    bt = math.gcd(n, _BT)

    x = x_nchw.reshape(n, 784)

    t1, t2, w1, b1v, b2v = _prep(conv1_w, conv2_w, fc1_w, conv1_b, conv2_b)
    w2 = fc2_w.astype(jnp.bfloat16)

    body = functools.partial(_fused_cnn_body, bt=bt)
    logits = pl.pallas_call(
        body,
        out_shape=jax.ShapeDtypeStruct((n, 10), jnp.float32),
        grid=(n // bt,),
        in_specs=[
            pl.BlockSpec((bt, 784), lambda i: (i, 0)),
            pl.BlockSpec((96, 512), lambda i: (0, 0)),
            pl.BlockSpec((1, 256), lambda i: (0, 0)),
            pl.BlockSpec((768, 512), lambda i: (0, 0)),
            pl.BlockSpec((1, 256), lambda i: (0, 0)),
            pl.BlockSpec((1792, 128), lambda i: (0, 0)),
            pl.BlockSpec((1, 128), lambda i: (0, 0)),
            pl.BlockSpec((128, 128), lambda i: (0, 0)),
            pl.BlockSpec((1, 128), lambda i: (0, 0)),
        ],
        out_specs=pl.BlockSpec((bt, 10), lambda i: (i, 0)),
        compiler_params=pltpu.CompilerParams(
            dimension_semantics=("parallel",),
            vmem_limit_bytes=100 * 1024 * 1024,
        ),
    )(x, t1, b1v, t2, b2v, w1, fc1_b.reshape(1, 128), w2,
      fc2_b.reshape(1, 128))
    return logits


# speedup vs baseline: 1.2822x; 1.2822x over previous
import functools
import math

import ml_dtypes
import numpy as np

import jax
import jax.numpy as jnp
from jax.experimental import pallas as pl
from jax.experimental.pallas import tpu as pltpu

_BT = 1024


def _masks1():
    m = np.zeros((9, 96, 512), np.float32)
    for dy in range(3):
        for dx in range(3):
            for w_out in range(28):
                w_in = w_out + dx - 1
                if 0 <= w_in < 28:
                    col = (w_out % 2) * 256 + (w_out // 2) * 16
                    m[dy * 3 + dx, dy * 32 + w_in, col:col + 16] = 1.0
    return m.reshape(864, 512).astype(ml_dtypes.bfloat16)


def _masks2():
    m = np.zeros((9, 768, 512), np.float32)
    for dy in range(3):
        for dx in range(3):
            for w_out in range(14):
                w_in = w_out + dx - 1
                if 0 <= w_in < 14:
                    col = (w_out % 2) * 256 + (w_out // 2) * 32
                    r = dy * 256 + w_in * 16
                    m[dy * 3 + dx, r:r + 16, col:col + 32] = 1.0
    return m.reshape(6912, 512).astype(ml_dtypes.bfloat16)


_M1 = _masks1()
_M2 = _masks2()


def _prep_body(c1w_ref, c2w_ref, f1w_ref, c1b_ref, c2b_ref, m1_ref, m2_ref,
               t1_ref, t2_ref, w1_ref, b1_ref, b2_ref):
    c1 = c1w_ref[...].astype(jnp.bfloat16)
    acc1 = jnp.zeros((96, 512), jnp.bfloat16)
    for tap in range(9):
        v = jnp.concatenate([c1[tap:tap + 1, :]] * 32, axis=1)
        acc1 = acc1 + m1_ref[96 * tap:96 * (tap + 1), :] * v
    t1_ref[...] = acc1

    c2 = c2w_ref[...].astype(jnp.bfloat16)
    acc2 = jnp.zeros((768, 512), jnp.bfloat16)
    for tap in range(9):
        blk = c2[16 * tap:16 * (tap + 1), :]
        w48 = jnp.concatenate([blk] * 48, axis=0)
        wb = jnp.concatenate([w48] * 16, axis=1)
        acc2 = acc2 + m2_ref[768 * tap:768 * (tap + 1), :] * wb
    t2_ref[...] = acc2

    f1 = f1w_ref[...].astype(jnp.bfloat16).reshape(7, 224, 128)
    f1 = jnp.concatenate([f1, jnp.zeros((7, 32, 128), jnp.bfloat16)], axis=1)
    w1_ref[...] = f1.reshape(1792, 128)

    b1_ref[...] = jnp.concatenate(
        [c1b_ref[...]] * 14 + [jnp.zeros((1, 32), jnp.float32)], axis=1)
    b2_ref[...] = jnp.concatenate(
        [c2b_ref[...]] * 7 + [jnp.zeros((1, 32), jnp.float32)], axis=1)


def _prep(conv1_w, conv2_w, fc1_w, conv1_b, conv2_b):
    return pl.pallas_call(
        _prep_body,
        out_shape=[
            jax.ShapeDtypeStruct((96, 512), jnp.bfloat16),
            jax.ShapeDtypeStruct((768, 512), jnp.bfloat16),
            jax.ShapeDtypeStruct((1792, 128), jnp.bfloat16),
            jax.ShapeDtypeStruct((1, 256), jnp.float32),
            jax.ShapeDtypeStruct((1, 256), jnp.float32),
        ],
        compiler_params=pltpu.CompilerParams(
            vmem_limit_bytes=64 * 1024 * 1024,
        ),
    )(conv1_w, conv2_w, fc1_w, conv1_b.reshape(1, 16), conv2_b.reshape(1, 32),
      jnp.asarray(_M1), jnp.asarray(_M2))


def _fused_cnn_body(x_ref, t1_ref, b1_ref, t2_ref, b2_ref, w1_ref, fb1_ref,
                    w2_ref, fb2_ref, o_ref, *, bt):
    x = x_ref[...]
    b1 = b1_ref[...]
    b2 = b2_ref[...]

    p1 = []
    for i in range(14):
        ya = jnp.dot(x[:, 64 * i:64 * i + 96], t1_ref[...],
                     preferred_element_type=jnp.float32)
        yb = jnp.dot(x[:, 64 * i + 32:64 * i + 128], t1_ref[...],
                     preferred_element_type=jnp.float32)
        m = jnp.maximum(ya, yb)
        m = jnp.maximum(m[:, :256], m[:, 256:])
        p1.append(jnp.maximum(m + b1, 0.0).astype(jnp.bfloat16))

    z256 = jnp.zeros((bt, 256), jnp.bfloat16)
    p1f = jnp.concatenate([z256] + p1 + [z256], axis=1)

    feats = []
    for i in range(7):
        ya = jnp.dot(p1f[:, 512 * i:512 * i + 768], t2_ref[...],
                     preferred_element_type=jnp.float32)
        yb = jnp.dot(p1f[:, 512 * i + 256:512 * i + 1024], t2_ref[...],
                     preferred_element_type=jnp.float32)
        m = jnp.maximum(ya, yb)
        m = jnp.maximum(m[:, :256], m[:, 256:])
        feats.append(jnp.maximum(m + b2, 0.0).astype(jnp.bfloat16))

    ff = jnp.concatenate(feats, axis=1)
    h = jnp.dot(ff, w1_ref[...], preferred_element_type=jnp.float32)
    h = jnp.maximum(h + fb1_ref[...], 0.0).astype(jnp.bfloat16)
    y = (jnp.dot(h, w2_ref[...], preferred_element_type=jnp.float32)
         + fb2_ref[...])
    o_ref[...] = y[:, :10]


def kernel(x_nchw, conv1_w, conv1_b, conv2_w, conv2_b, fc1_w, fc1_b,
           fc2_w, fc2_b):
    n = x_nchw.shape[0]
    bt = math.gcd(n, _BT)

    x = jnp.pad(x_nchw.reshape(n, 28, 28), ((0, 0), (1, 1), (0, 4)))
    x = x.reshape(n, 960).astype(jnp.bfloat16)

    t1, t2, w1, b1v, b2v = _prep(conv1_w, conv2_w, fc1_w, conv1_b, conv2_b)
    w2 = fc2_w.astype(jnp.bfloat16)

    body = functools.partial(_fused_cnn_body, bt=bt)
    logits = pl.pallas_call(
        body,
        out_shape=jax.ShapeDtypeStruct((n, 10), jnp.float32),
        grid=(n // bt,),
        in_specs=[
            pl.BlockSpec((bt, 960), lambda i: (i, 0)),
            pl.BlockSpec((96, 512), lambda i: (0, 0)),
            pl.BlockSpec((1, 256), lambda i: (0, 0)),
            pl.BlockSpec((768, 512), lambda i: (0, 0)),
            pl.BlockSpec((1, 256), lambda i: (0, 0)),
            pl.BlockSpec((1792, 128), lambda i: (0, 0)),
            pl.BlockSpec((1, 128), lambda i: (0, 0)),
            pl.BlockSpec((128, 128), lambda i: (0, 0)),
            pl.BlockSpec((1, 128), lambda i: (0, 0)),
        ],
        out_specs=pl.BlockSpec((bt, 10), lambda i: (i, 0)),
        compiler_params=pltpu.CompilerParams(
            dimension_semantics=("parallel",),
            vmem_limit_bytes=100 * 1024 * 1024,
        ),
    )(x, t1, b1v, t2, b2v, w1, fc1_b.reshape(1, 128), w2,
      fc2_b.reshape(1, 128))
    return logits
